# baseline (speedup 1.0000x reference)
import functools

import jax
import jax.numpy as jnp
from jax import lax
from jax.experimental import pallas as pl
from jax.experimental.pallas import tpu as pltpu
from jax.experimental.pallas import tpu_sc as plsc

_B = 4096
_D = 128
_LANES = 16
_CHUNKS = _D // _LANES
_PIPE = 4


def _make_loss_kernel(num_workers: int, b_per_w: int):
    mesh = plsc.VectorSubcoreMesh(core_axis_name="c", subcore_axis_name="s")
    rows = b_per_w // _PIPE

    @functools.partial(
        pl.kernel,
        mesh=mesh,
        out_type=jax.ShapeDtypeStruct((num_workers, _LANES), jnp.float32),
        scratch_types=[
            pltpu.VMEM((b_per_w,), jnp.int32),
            pltpu.VMEM((b_per_w,), jnp.int32),
            pltpu.VMEM((b_per_w,), jnp.int32),
            pltpu.VMEM((b_per_w, _LANES), jnp.float32),
            pltpu.VMEM((b_per_w, _D), jnp.float32),
            pltpu.VMEM((b_per_w, _D), jnp.float32),
            pltpu.VMEM((b_per_w, _D), jnp.float32),
            pltpu.VMEM((_LANES,), jnp.float32),
            pltpu.SemaphoreType.DMA,
        ] + [pltpu.SemaphoreType.DMA] * (2 * _PIPE),
    )
    def loss_kernel(h_hbm, r_hbm, t_hbm, v_hbm, ent_hbm, rel_hbm, out_hbm,
                    h_idx, r_idx, t_idx, v_vm, h_rows, r_rows, t_rows,
                    acc_vm, sem_v, *sems):
        sem_idx = sems[:_PIPE]
        sem_g = sems[_PIPE:]
        num_cores = lax.axis_size("c")
        wid = lax.axis_index("s") * num_cores + lax.axis_index("c")
        base = wid * b_per_w

        cp_idx = []
        for c in range(_PIPE):
            sl_vm = pl.ds(c * rows, rows)
            sl_hbm = pl.ds(base + c * rows, rows)
            cp_idx.append([
                pltpu.async_copy(h_hbm.at[sl_hbm], h_idx.at[sl_vm],
                                 sem_idx[c]),
                pltpu.async_copy(r_hbm.at[sl_hbm], r_idx.at[sl_vm],
                                 sem_idx[c]),
                pltpu.async_copy(t_hbm.at[sl_hbm], t_idx.at[sl_vm],
                                 sem_idx[c]),
            ])
        cp_v = pltpu.async_copy(v_hbm.at[pl.ds(base, b_per_w)], v_vm, sem_v)

        gathers = []
        for c in range(_PIPE):
            sl = pl.ds(c * rows, rows)
            for cp in cp_idx[c]:
                cp.wait()
            gathers.append((
                pltpu.async_copy(ent_hbm.at[h_idx.at[sl]],
                                 h_rows.at[sl], sem_g[c]),
                pltpu.async_copy(rel_hbm.at[r_idx.at[sl]],
                                 r_rows.at[sl], sem_g[c]),
                pltpu.async_copy(ent_hbm.at[t_idx.at[sl]],
                                 t_rows.at[sl], sem_g[c]),
            ))

        def body(i, acc):
            vv = v_vm[i, :]
            dd = jnp.zeros((_LANES,), jnp.float32)
            for k in range(_CHUNKS):
                sl = pl.ds(k * _LANES, _LANES)
                d = h_rows[i, sl] + r_rows[i, sl] - t_rows[i, sl]
                dd = dd + d * d
            return acc + dd * vv

        cp_v.wait()
        acc = jnp.zeros((_LANES,), jnp.float32)
        for c in range(_PIPE):
            for cp in gathers[c]:
                cp.wait()
            acc = plsc.parallel_loop(c * rows, (c + 1) * rows, unroll=8,
                                     carry=acc)(body)

        acc_vm[...] = acc
        pltpu.sync_copy(acc_vm, out_hbm.at[wid])

    return loss_kernel


def kernel(h, r, t, v, adj, ent_emb, rel_emb, W, b):
    info = plsc.get_sparse_core_info()
    num_workers = info.num_cores * info.num_subcores
    b_per_w = _B // num_workers
    loss_kernel = _make_loss_kernel(num_workers, b_per_w)
    v_rep = jnp.broadcast_to(v.astype(jnp.float32)[:, None], (_B, _LANES))
    partials = loss_kernel(
        h.astype(jnp.int32), r.astype(jnp.int32), t.astype(jnp.int32),
        v_rep, ent_emb, rel_emb)
    return jnp.sum(partials) / jnp.float32(_B)


# speedup vs baseline: 1.0086x; 1.0086x over previous
import functools

import jax
import jax.numpy as jnp
from jax import lax
from jax.experimental import pallas as pl
from jax.experimental.pallas import tpu as pltpu
from jax.experimental.pallas import tpu_sc as plsc

_B = 4096
_D = 128
_LANES = 16
_CHUNKS = _D // _LANES
_PIPE = 2


def _make_loss_kernel(num_workers: int, b_per_w: int):
    mesh = plsc.VectorSubcoreMesh(core_axis_name="c", subcore_axis_name="s")
    rows = b_per_w // _PIPE

    @functools.partial(
        pl.kernel,
        mesh=mesh,
        out_type=jax.ShapeDtypeStruct((num_workers, _LANES), jnp.float32),
        scratch_types=[
            pltpu.VMEM((b_per_w,), jnp.int32),
            pltpu.VMEM((b_per_w,), jnp.int32),
            pltpu.VMEM((b_per_w,), jnp.int32),
            pltpu.VMEM((b_per_w, _LANES), jnp.float32),
            pltpu.VMEM((b_per_w, _D), jnp.float32),
            pltpu.VMEM((b_per_w, _D), jnp.float32),
            pltpu.VMEM((b_per_w, _D), jnp.float32),
            pltpu.VMEM((_LANES,), jnp.float32),
            pltpu.SemaphoreType.DMA,
        ] + [pltpu.SemaphoreType.DMA] * (2 * _PIPE),
    )
    def loss_kernel(h_hbm, r_hbm, t_hbm, v_hbm, ent_hbm, rel_hbm, out_hbm,
                    h_idx, r_idx, t_idx, v_vm, h_rows, r_rows, t_rows,
                    acc_vm, sem_v, *sems):
        sem_idx = sems[:_PIPE]
        sem_g = sems[_PIPE:]
        num_cores = lax.axis_size("c")
        wid = lax.axis_index("s") * num_cores + lax.axis_index("c")
        base = wid * b_per_w

        cp_idx = []
        for c in range(_PIPE):
            sl_vm = pl.ds(c * rows, rows)
            sl_hbm = pl.ds(base + c * rows, rows)
            cp_idx.append([
                pltpu.async_copy(h_hbm.at[sl_hbm], h_idx.at[sl_vm],
                                 sem_idx[c]),
                pltpu.async_copy(r_hbm.at[sl_hbm], r_idx.at[sl_vm],
                                 sem_idx[c]),
                pltpu.async_copy(t_hbm.at[sl_hbm], t_idx.at[sl_vm],
                                 sem_idx[c]),
            ])
        cp_v = pltpu.async_copy(v_hbm.at[pl.ds(base, b_per_w)], v_vm, sem_v)

        gathers = []
        for c in range(_PIPE):
            sl = pl.ds(c * rows, rows)
            for cp in cp_idx[c]:
                cp.wait()
            gathers.append((
                pltpu.async_copy(ent_hbm.at[h_idx.at[sl]],
                                 h_rows.at[sl], sem_g[c]),
                pltpu.async_copy(rel_hbm.at[r_idx.at[sl]],
                                 r_rows.at[sl], sem_g[c]),
                pltpu.async_copy(ent_hbm.at[t_idx.at[sl]],
                                 t_rows.at[sl], sem_g[c]),
            ))

        def body(i, acc):
            vv = v_vm[i, :]
            dd0 = jnp.zeros((_LANES,), jnp.float32)
            dd1 = jnp.zeros((_LANES,), jnp.float32)
            for k in range(0, _CHUNKS, 2):
                sl0 = pl.ds(k * _LANES, _LANES)
                sl1 = pl.ds((k + 1) * _LANES, _LANES)
                d0 = h_rows[i, sl0] + r_rows[i, sl0] - t_rows[i, sl0]
                d1 = h_rows[i, sl1] + r_rows[i, sl1] - t_rows[i, sl1]
                dd0 = dd0 + d0 * d0
                dd1 = dd1 + d1 * d1
            return acc + (dd0 + dd1) * vv

        cp_v.wait()
        acc = jnp.zeros((_LANES,), jnp.float32)
        for c in range(_PIPE):
            for cp in gathers[c]:
                cp.wait()
            acc = plsc.parallel_loop(c * rows, (c + 1) * rows, unroll=8,
                                     carry=acc)(body)

        acc_vm[...] = acc
        pltpu.sync_copy(acc_vm, out_hbm.at[wid])

    return loss_kernel


def kernel(h, r, t, v, adj, ent_emb, rel_emb, W, b):
    info = plsc.get_sparse_core_info()
    num_workers = info.num_cores * info.num_subcores
    b_per_w = _B // num_workers
    loss_kernel = _make_loss_kernel(num_workers, b_per_w)
    v_rep = jnp.broadcast_to(v.astype(jnp.float32)[:, None], (_B, _LANES))
    partials = loss_kernel(
        h.astype(jnp.int32), r.astype(jnp.int32), t.astype(jnp.int32),
        v_rep, ent_emb, rel_emb)
    return jnp.sum(partials) / jnp.float32(_B)


# speedup vs baseline: 1.0214x; 1.0128x over previous
import functools

import jax
import jax.numpy as jnp
from jax import lax
from jax.experimental import pallas as pl
from jax.experimental.pallas import tpu as pltpu
from jax.experimental.pallas import tpu_sc as plsc

_B = 4096
_D = 128
_LANES = 16
_CHUNKS = _D // _LANES
_PIPE = 2


def _make_loss_kernel(num_workers: int, b_per_w: int):
    mesh = plsc.VectorSubcoreMesh(core_axis_name="c", subcore_axis_name="s")
    rows = b_per_w // _PIPE

    @functools.partial(
        pl.kernel,
        mesh=mesh,
        out_type=jax.ShapeDtypeStruct((num_workers, _LANES), jnp.float32),
        scratch_types=[
            pltpu.VMEM((b_per_w,), jnp.int32),
            pltpu.VMEM((b_per_w,), jnp.int32),
            pltpu.VMEM((b_per_w,), jnp.int32),
            pltpu.VMEM((b_per_w, _LANES), jnp.float32),
            pltpu.VMEM((b_per_w, _D), jnp.float32),
            pltpu.VMEM((b_per_w, _D), jnp.float32),
            pltpu.VMEM((b_per_w, _D), jnp.float32),
            pltpu.VMEM((_LANES,), jnp.float32),
            pltpu.SemaphoreType.DMA,
        ] + [pltpu.SemaphoreType.DMA] * (2 * _PIPE),
    )
    def loss_kernel(h_hbm, r_hbm, t_hbm, v_hbm, ent_hbm, rel_hbm, out_hbm,
                    h_idx, r_idx, t_idx, v_vm, h_rows, r_rows, t_rows,
                    acc_vm, sem_v, *sems):
        sem_idx = sems[:_PIPE]
        sem_g = sems[_PIPE:]
        num_cores = lax.axis_size("c")
        wid = lax.axis_index("s") * num_cores + lax.axis_index("c")
        base = wid * b_per_w

        cp_idx = []
        for c in range(_PIPE):
            sl_vm = pl.ds(c * rows, rows)
            sl_hbm = pl.ds(base + c * rows, rows)
            cp_idx.append([
                pltpu.async_copy(h_hbm.at[sl_hbm], h_idx.at[sl_vm],
                                 sem_idx[c]),
                pltpu.async_copy(r_hbm.at[sl_hbm], r_idx.at[sl_vm],
                                 sem_idx[c]),
                pltpu.async_copy(t_hbm.at[sl_hbm], t_idx.at[sl_vm],
                                 sem_idx[c]),
            ])
        cp_v = pltpu.async_copy(v_hbm.at[pl.ds(base, b_per_w)], v_vm, sem_v)

        gathers = []
        for c in range(_PIPE):
            sl = pl.ds(c * rows, rows)
            for cp in cp_idx[c]:
                cp.wait()
            gathers.append((
                pltpu.async_copy(ent_hbm.at[h_idx.at[sl]],
                                 h_rows.at[sl], sem_g[c]),
                pltpu.async_copy(rel_hbm.at[r_idx.at[sl]],
                                 r_rows.at[sl], sem_g[c]),
                pltpu.async_copy(ent_hbm.at[t_idx.at[sl]],
                                 t_rows.at[sl], sem_g[c]),
            ))

        def body(i, acc):
            vv = v_vm[i, :]
            dd = jnp.zeros((_LANES,), jnp.float32)
            for k in range(_CHUNKS):
                sl = pl.ds(k * _LANES, _LANES)
                d = h_rows[i, sl] + r_rows[i, sl] - t_rows[i, sl]
                dd = dd + d * d
            return acc + dd * vv

        cp_v.wait()
        acc = jnp.zeros((_LANES,), jnp.float32)
        for c in range(_PIPE):
            for cp in gathers[c]:
                cp.wait()
            acc = plsc.parallel_loop(c * rows, (c + 1) * rows, unroll=8,
                                     carry=acc)(body)

        acc_vm[...] = acc
        pltpu.sync_copy(acc_vm, out_hbm.at[wid])

    return loss_kernel


def kernel(h, r, t, v, adj, ent_emb, rel_emb, W, b):
    info = plsc.get_sparse_core_info()
    num_workers = info.num_cores * info.num_subcores
    b_per_w = _B // num_workers
    loss_kernel = _make_loss_kernel(num_workers, b_per_w)
    v_rep = jnp.broadcast_to(v.astype(jnp.float32)[:, None], (_B, _LANES))
    partials = loss_kernel(
        h.astype(jnp.int32), r.astype(jnp.int32), t.astype(jnp.int32),
        v_rep, ent_emb, rel_emb)
    return jnp.sum(partials) / jnp.float32(_B)
